# baseline (speedup 1.0000x reference)
import functools

import jax
import jax.numpy as jnp
from jax import lax
from jax.experimental import pallas as pl
from jax.experimental.pallas import tpu as pltpu
from jax.experimental.pallas import tpu_sc as plsc

BATCH = 16384
EMB_DIM = 16
IN_FEAT = 6
HIDDEN = 64


def _sc_gather(client_emb, segment_emb, currency_emb, cid, sid, uid):
    info = plsc.get_sparse_core_info()
    NC, NS = info.num_cores, info.num_subcores
    NW = NC * NS
    bpw = BATCH // NW

    mesh = plsc.VectorSubcoreMesh(core_axis_name="c", subcore_axis_name="s")

    @functools.partial(
        pl.kernel,
        mesh=mesh,
        out_type=[jax.ShapeDtypeStruct((BATCH, EMB_DIM), jnp.float32)] * 3,
        scratch_types=[
            pltpu.VMEM((bpw,), jnp.int32),
            pltpu.VMEM((bpw,), jnp.int32),
            pltpu.VMEM((bpw,), jnp.int32),
            pltpu.VMEM((bpw, EMB_DIM), jnp.float32),
            pltpu.VMEM((bpw, EMB_DIM), jnp.float32),
            pltpu.VMEM((bpw, EMB_DIM), jnp.float32),
            pltpu.SemaphoreType.DMA,
            pltpu.SemaphoreType.DMA,
            pltpu.SemaphoreType.DMA,
        ],
        compiler_params=pltpu.CompilerParams(use_tc_tiling_on_sc=False),
    )
    def k(ce, se, ue, ci, si, ui, oc, osg, ocu,
          iv0, iv1, iv2, rv0, rv1, rv2, s0, s1, s2):
        wid = lax.axis_index("s") * NC + lax.axis_index("c")
        base = wid * bpw
        pltpu.sync_copy(ci.at[pl.ds(base, bpw)], iv0)
        pltpu.sync_copy(si.at[pl.ds(base, bpw)], iv1)
        pltpu.sync_copy(ui.at[pl.ds(base, bpw)], iv2)
        c0 = pltpu.async_copy(ce.at[iv0], rv0, s0)
        c1 = pltpu.async_copy(se.at[iv1], rv1, s1)
        c2 = pltpu.async_copy(ue.at[iv2], rv2, s2)
        c0.wait()
        c1.wait()
        c2.wait()
        pltpu.sync_copy(rv0, oc.at[pl.ds(base, bpw)])
        pltpu.sync_copy(rv1, osg.at[pl.ds(base, bpw)])
        pltpu.sync_copy(rv2, ocu.at[pl.ds(base, bpw)])

    return k(client_emb, segment_emb, currency_emb, cid, sid, uid)


def _mlp_body(f_ref, c_ref, s_ref, u_ref, w1_ref, b1_ref, w2_ref, b2_ref,
              o_ref):
    h = jnp.dot(f_ref[...], w1_ref[0:IN_FEAT, :],
                preferred_element_type=jnp.float32)
    h += jnp.dot(c_ref[...], w1_ref[IN_FEAT:IN_FEAT + EMB_DIM, :],
                 preferred_element_type=jnp.float32)
    h += jnp.dot(s_ref[...], w1_ref[IN_FEAT + EMB_DIM:IN_FEAT + 2 * EMB_DIM, :],
                 preferred_element_type=jnp.float32)
    h += jnp.dot(u_ref[...], w1_ref[IN_FEAT + 2 * EMB_DIM:, :],
                 preferred_element_type=jnp.float32)
    h = jnp.maximum(h + b1_ref[...], 0.0)
    o_ref[...] = jnp.dot(h, w2_ref[...],
                         preferred_element_type=jnp.float32) + b2_ref[...]


def _mlp(features, c_emb, s_emb, u_emb, W1, b1, W2, b2):
    BLK = 2048
    grid = (BATCH // BLK,)
    d_in = IN_FEAT + 3 * EMB_DIM
    out = pl.pallas_call(
        _mlp_body,
        grid=grid,
        in_specs=[
            pl.BlockSpec((BLK, IN_FEAT), lambda i: (i, 0)),
            pl.BlockSpec((BLK, EMB_DIM), lambda i: (i, 0)),
            pl.BlockSpec((BLK, EMB_DIM), lambda i: (i, 0)),
            pl.BlockSpec((BLK, EMB_DIM), lambda i: (i, 0)),
            pl.BlockSpec((d_in, HIDDEN), lambda i: (0, 0)),
            pl.BlockSpec((1, HIDDEN), lambda i: (0, 0)),
            pl.BlockSpec((HIDDEN, 1), lambda i: (0, 0)),
            pl.BlockSpec((1, 1), lambda i: (0, 0)),
        ],
        out_specs=pl.BlockSpec((BLK, 1), lambda i: (i, 0)),
        out_shape=jax.ShapeDtypeStruct((BATCH, 1), jnp.float32),
    )(features, c_emb, s_emb, u_emb, W1, b1.reshape(1, HIDDEN), W2,
      b2.reshape(1, 1))
    return out[:, 0]


def kernel(features, client_id, segment_id, currency_pair_id,
           client_emb, segment_emb, currency_emb, W1, b1, W2, b2):
    cid = client_id.astype(jnp.int32)
    sid = segment_id.astype(jnp.int32)
    uid = currency_pair_id.astype(jnp.int32)
    c_emb, s_emb, u_emb = _sc_gather(client_emb, segment_emb, currency_emb,
                                     cid, sid, uid)
    return _mlp(features, c_emb, s_emb, u_emb, W1, b1, W2, b2)
